# baseline (speedup 1.0000x reference)
import functools

import jax
import jax.numpy as jnp
from jax import lax
from jax.experimental import pallas as pl
from jax.experimental.pallas import tpu as pltpu

try:
    from jax.experimental.pallas import tpu_sc as plsc
    _info = plsc.get_sparse_core_info()
    _NC, _NS = _info.num_cores, _info.num_subcores
except Exception:
    plsc = None
    _NC, _NS = 2, 16

_B = 16384
_UEMB = 64
_NUSERS = 1000000
_NW = _NC * _NS
_BPW = _B // _NW

_TN = 4096


def _transpose_body(int_ref, out_ref):
    out_ref[...] = int_ref[...].T


def _tc_transpose(tabt):
    grid = (pl.cdiv(_NUSERS, _TN),)
    return pl.pallas_call(
        _transpose_body,
        grid=grid,
        in_specs=[pl.BlockSpec((_UEMB, _TN), lambda i: (0, i))],
        out_specs=pl.BlockSpec((_TN, _UEMB), lambda i: (i, 0)),
        out_shape=jax.ShapeDtypeStruct((_NUSERS, _UEMB), jnp.float32),
    )(tabt)


_G = 16
_NGRP = _BPW // _G
_LAG = 2


def _make_sc_gather():
    mesh = plsc.VectorSubcoreMesh(core_axis_name="c", subcore_axis_name="s")

    @functools.partial(
        pl.kernel,
        mesh=mesh,
        out_type=jax.ShapeDtypeStruct((_B, _UEMB), jnp.float32),
        scratch_types=[
            pltpu.VMEM((_BPW,), jnp.int32),
            pltpu.VMEM((_BPW, _UEMB), jnp.float32),
            pltpu.SemaphoreType.DMA,
        ],
        compiler_params=pltpu.CompilerParams(use_tc_tiling_on_sc=True,
                                             needs_layout_passes=False),
    )
    def sc_gather(table_hbm, idx_hbm, out_hbm, idx_v, rows_v, sem):
        wid = lax.axis_index("s") * _NC + lax.axis_index("c")
        base = wid * _BPW
        lanes = lax.iota(jnp.int32, 16)
        pltpu.sync_copy(idx_hbm.at[pl.ds(base, _BPW)], idx_v)

        def fire(g):
            v = idx_v[pl.ds(g * _G, _G)]
            for j in range(_G):
                u = lax.reduce_max(jnp.where(lanes == j, v, -1), (0,))
                t = lax.shift_right_logical(u, 3)
                s = jnp.bitwise_and(u, 7)
                pltpu.async_copy(table_hbm.at[t, s], rows_v.at[g * _G + j],
                                 sem)

        def drain():
            for j in range(_G):
                pltpu.make_async_copy(table_hbm.at[0, 0], rows_v.at[0],
                                      sem).wait()

        def body(g, carry):
            fire(g)

            @pl.when(g >= _LAG)
            def _():
                drain()

            return carry

        lax.fori_loop(0, _NGRP, body, 0)
        for _ in range(_LAG):
            drain()
        pltpu.sync_copy(rows_v, out_hbm.at[pl.ds(base, _BPW)])

    return sc_gather


def _tc_body(euid_ref, city_ref, d_ref, ts_ref, tc_ref, poit_ref,
             citywt_ref, dayw_ref, dayb_ref, timew_ref, timeb_ref,
             poiw_ref, poib_ref, out_ref):
    out_ref[0:_UEMB, :] = euid_ref[...].T
    cityv = city_ref[...]
    citywt = citywt_ref[...]
    e_city = citywt[:, 0:1] * (cityv == 0).astype(jnp.float32)
    for c in range(1, 4):
        e_city = e_city + citywt[:, c:c + 1] * (cityv == c).astype(
            jnp.float32)
    out_ref[64:68, :] = e_city
    out_ref[68:76, :] = jnp.maximum(
        dayw_ref[...] * d_ref[...] + dayb_ref[...], 0.0)
    out_ref[76:84, :] = jnp.maximum(
        timew_ref[:, 0:1] * ts_ref[...] + timew_ref[:, 1:2] * tc_ref[...]
        + timeb_ref[...], 0.0)
    out_ref[84:94, :] = jnp.maximum(
        jnp.dot(poiw_ref[...], poit_ref[...],
                preferred_element_type=jnp.float32) + poib_ref[...], 0.0)


def _tc_dense(e_uid, city1, d1, ts1, tc1, poit,
              citywt, dayw, dayb, timew, timeb, poiw, poib):
    bn = 2048
    grid = (_B // bn,)
    col = lambda i: (0, i)
    row = lambda i: (i, 0)
    rep = lambda i: (0, 0)
    return pl.pallas_call(
        _tc_body,
        grid=grid,
        in_specs=[
            pl.BlockSpec((bn, _UEMB), row),
            pl.BlockSpec((1, bn), col),
            pl.BlockSpec((1, bn), col),
            pl.BlockSpec((1, bn), col),
            pl.BlockSpec((1, bn), col),
            pl.BlockSpec((85, bn), col),
            pl.BlockSpec((4, 4), rep),
            pl.BlockSpec((8, 1), rep),
            pl.BlockSpec((8, 1), rep),
            pl.BlockSpec((8, 2), rep),
            pl.BlockSpec((8, 1), rep),
            pl.BlockSpec((10, 85), rep),
            pl.BlockSpec((10, 1), rep),
        ],
        out_specs=pl.BlockSpec((94, bn), col),
        out_shape=jax.ShapeDtypeStruct((94, _B), jnp.float32),
    )(e_uid, city1, d1, ts1, tc1, poit,
      citywt, dayw, dayb, timew, timeb, poiw, poib)


def kernel(uid, d_norm, t_sin, t_cos, city, poi_norm,
           uid_emb_W, city_emb_W, day_W, day_b, time_W, time_b,
           poi_W, poi_b):
    table_rm = _tc_transpose(uid_emb_W.T)
    table3 = table_rm.reshape(_NUSERS // 8, 8, _UEMB)
    e_uid = _make_sc_gather()(table3, uid.astype(jnp.int32))
    outt = _tc_dense(
        e_uid,
        city.astype(jnp.int32).reshape(1, _B),
        d_norm.reshape(1, _B),
        t_sin.reshape(1, _B),
        t_cos.reshape(1, _B),
        poi_norm.T,
        city_emb_W.T,
        day_W,
        day_b.reshape(8, 1),
        time_W,
        time_b.reshape(8, 1),
        poi_W,
        poi_b.reshape(10, 1),
    )
    return outt.T


# speedup vs baseline: 1.3237x; 1.3237x over previous
import functools

import jax
import jax.numpy as jnp
from jax import lax
from jax.experimental import pallas as pl
from jax.experimental.pallas import tpu as pltpu

try:
    from jax.experimental.pallas import tpu_sc as plsc
    _info = plsc.get_sparse_core_info()
    _NC, _NS = _info.num_cores, _info.num_subcores
except Exception:
    plsc = None
    _NC, _NS = 2, 16

_B = 16384
_UEMB = 64
_NUSERS = 1000000
_NW = _NC * _NS
_BPW = _B // _NW

_TN = 16384


def _transpose_body(int_ref, out_ref):
    out_ref[...] = int_ref[...].T


def _tc_transpose(tabt):
    grid = (pl.cdiv(_NUSERS, _TN),)
    return pl.pallas_call(
        _transpose_body,
        grid=grid,
        in_specs=[pl.BlockSpec((_UEMB, _TN), lambda i: (0, i))],
        out_specs=pl.BlockSpec((_TN, _UEMB), lambda i: (i, 0)),
        out_shape=jax.ShapeDtypeStruct((_NUSERS, _UEMB), jnp.float32),
    )(tabt)


_G = 16
_NGRP = _BPW // _G
_LAG = 2


def _make_sc_gather():
    mesh = plsc.VectorSubcoreMesh(core_axis_name="c", subcore_axis_name="s")

    @functools.partial(
        pl.kernel,
        mesh=mesh,
        out_type=jax.ShapeDtypeStruct((_B, _UEMB), jnp.float32),
        scratch_types=[
            pltpu.VMEM((_BPW,), jnp.int32),
            pltpu.VMEM((_BPW, _UEMB), jnp.float32),
            pltpu.SemaphoreType.DMA,
        ],
        compiler_params=pltpu.CompilerParams(use_tc_tiling_on_sc=True,
                                             needs_layout_passes=False),
    )
    def sc_gather(table_hbm, idx_hbm, out_hbm, idx_v, rows_v, sem):
        wid = lax.axis_index("s") * _NC + lax.axis_index("c")
        base = wid * _BPW
        lanes = lax.iota(jnp.int32, 16)
        pltpu.sync_copy(idx_hbm.at[pl.ds(base, _BPW)], idx_v)

        def fire(g):
            v = idx_v[pl.ds(g * _G, _G)]
            for j in range(_G):
                u = lax.reduce_max(jnp.where(lanes == j, v, -1), (0,))
                t = lax.shift_right_logical(u, 3)
                s = jnp.bitwise_and(u, 7)
                pltpu.async_copy(table_hbm.at[t, s], rows_v.at[g * _G + j],
                                 sem)

        def drain():
            for j in range(_G):
                pltpu.make_async_copy(table_hbm.at[0, 0], rows_v.at[0],
                                      sem).wait()

        def body(g, carry):
            fire(g)

            @pl.when(g >= _LAG)
            def _():
                drain()

            return carry

        lax.fori_loop(0, _NGRP, body, 0)
        for _ in range(_LAG):
            drain()
        pltpu.sync_copy(rows_v, out_hbm.at[pl.ds(base, _BPW)])

    return sc_gather


def _tc_body(euid_ref, city_ref, d_ref, ts_ref, tc_ref, poit_ref,
             citywt_ref, dayw_ref, dayb_ref, timew_ref, timeb_ref,
             poiw_ref, poib_ref, out_ref):
    out_ref[0:_UEMB, :] = euid_ref[...].T
    cityv = city_ref[...]
    citywt = citywt_ref[...]
    e_city = citywt[:, 0:1] * (cityv == 0).astype(jnp.float32)
    for c in range(1, 4):
        e_city = e_city + citywt[:, c:c + 1] * (cityv == c).astype(
            jnp.float32)
    out_ref[64:68, :] = e_city
    out_ref[68:76, :] = jnp.maximum(
        dayw_ref[...] * d_ref[...] + dayb_ref[...], 0.0)
    out_ref[76:84, :] = jnp.maximum(
        timew_ref[:, 0:1] * ts_ref[...] + timew_ref[:, 1:2] * tc_ref[...]
        + timeb_ref[...], 0.0)
    out_ref[84:94, :] = jnp.maximum(
        jnp.dot(poiw_ref[...], poit_ref[...],
                preferred_element_type=jnp.float32) + poib_ref[...], 0.0)


def _tc_dense(e_uid, city1, d1, ts1, tc1, poit,
              citywt, dayw, dayb, timew, timeb, poiw, poib):
    bn = 2048
    grid = (_B // bn,)
    col = lambda i: (0, i)
    row = lambda i: (i, 0)
    rep = lambda i: (0, 0)
    return pl.pallas_call(
        _tc_body,
        grid=grid,
        in_specs=[
            pl.BlockSpec((bn, _UEMB), row),
            pl.BlockSpec((1, bn), col),
            pl.BlockSpec((1, bn), col),
            pl.BlockSpec((1, bn), col),
            pl.BlockSpec((1, bn), col),
            pl.BlockSpec((85, bn), col),
            pl.BlockSpec((4, 4), rep),
            pl.BlockSpec((8, 1), rep),
            pl.BlockSpec((8, 1), rep),
            pl.BlockSpec((8, 2), rep),
            pl.BlockSpec((8, 1), rep),
            pl.BlockSpec((10, 85), rep),
            pl.BlockSpec((10, 1), rep),
        ],
        out_specs=pl.BlockSpec((94, bn), col),
        out_shape=jax.ShapeDtypeStruct((94, _B), jnp.float32),
    )(e_uid, city1, d1, ts1, tc1, poit,
      citywt, dayw, dayb, timew, timeb, poiw, poib)


def kernel(uid, d_norm, t_sin, t_cos, city, poi_norm,
           uid_emb_W, city_emb_W, day_W, day_b, time_W, time_b,
           poi_W, poi_b):
    table_rm = _tc_transpose(uid_emb_W.T)
    table3 = table_rm.reshape(_NUSERS // 8, 8, _UEMB)
    e_uid = _make_sc_gather()(table3, uid.astype(jnp.int32))
    outt = _tc_dense(
        e_uid,
        city.astype(jnp.int32).reshape(1, _B),
        d_norm.reshape(1, _B),
        t_sin.reshape(1, _B),
        t_cos.reshape(1, _B),
        poi_norm.T,
        city_emb_W.T,
        day_W,
        day_b.reshape(8, 1),
        time_W,
        time_b.reshape(8, 1),
        poi_W,
        poi_b.reshape(10, 1),
    )
    return outt.T


# speedup vs baseline: 1.3243x; 1.0004x over previous
import functools

import jax
import jax.numpy as jnp
from jax import lax
from jax.experimental import pallas as pl
from jax.experimental.pallas import tpu as pltpu

try:
    from jax.experimental.pallas import tpu_sc as plsc
    _info = plsc.get_sparse_core_info()
    _NC, _NS = _info.num_cores, _info.num_subcores
except Exception:
    plsc = None
    _NC, _NS = 2, 16

_B = 16384
_UEMB = 64
_NUSERS = 1000000
_NW = _NC * _NS
_BPW = _B // _NW

_TN = 16384


def _transpose_body(int_ref, out_ref):
    out_ref[:, 0:_UEMB] = int_ref[...].T


def _tc_transpose(tabt):
    grid = (pl.cdiv(_NUSERS, _TN),)
    return pl.pallas_call(
        _transpose_body,
        grid=grid,
        in_specs=[pl.BlockSpec((_UEMB, _TN), lambda i: (0, i))],
        out_specs=pl.BlockSpec((_TN, 128), lambda i: (i, 0)),
        out_shape=jax.ShapeDtypeStruct((_NUSERS, 128), jnp.float32),
    )(tabt)


_G = 16
_NGRP = _BPW // _G
_LAG = 2


def _make_sc_gather():
    mesh = plsc.VectorSubcoreMesh(core_axis_name="c", subcore_axis_name="s")

    @functools.partial(
        pl.kernel,
        mesh=mesh,
        out_type=jax.ShapeDtypeStruct((_B, 128), jnp.float32),
        scratch_types=[
            pltpu.VMEM((_BPW,), jnp.int32),
            pltpu.VMEM((_BPW, 128), jnp.float32),
            pltpu.SemaphoreType.DMA,
        ],
        compiler_params=pltpu.CompilerParams(use_tc_tiling_on_sc=True,
                                             needs_layout_passes=False),
    )
    def sc_gather(table_hbm, idx_hbm, out_hbm, idx_v, rows_v, sem):
        wid = lax.axis_index("s") * _NC + lax.axis_index("c")
        base = wid * _BPW
        lanes = lax.iota(jnp.int32, 16)
        pltpu.sync_copy(idx_hbm.at[pl.ds(base, _BPW)], idx_v)

        def fire(g):
            v = idx_v[pl.ds(g * _G, _G)]
            for j in range(_G):
                u = lax.reduce_max(jnp.where(lanes == j, v, -1), (0,))
                t = lax.shift_right_logical(u, 3)
                s = jnp.bitwise_and(u, 7)
                pltpu.async_copy(table_hbm.at[t, s], rows_v.at[g * _G + j],
                                 sem)

        def drain():
            for j in range(_G):
                pltpu.make_async_copy(table_hbm.at[0, 0], rows_v.at[0],
                                      sem).wait()

        def body(g, carry):
            fire(g)

            @pl.when(g >= _LAG)
            def _():
                drain()

            return carry

        lax.fori_loop(0, _NGRP, body, 0)
        for _ in range(_LAG):
            drain()
        pltpu.sync_copy(rows_v, out_hbm.at[pl.ds(base, _BPW)])

    return sc_gather


def _tc_body(euid_ref, city_ref, d_ref, ts_ref, tc_ref, poit_ref,
             citywt_ref, dayw_ref, dayb_ref, timew_ref, timeb_ref,
             poiw_ref, poib_ref, out_ref):
    out_ref[0:_UEMB, :] = euid_ref[:, 0:_UEMB].T
    cityv = city_ref[...]
    citywt = citywt_ref[...]
    e_city = citywt[:, 0:1] * (cityv == 0).astype(jnp.float32)
    for c in range(1, 4):
        e_city = e_city + citywt[:, c:c + 1] * (cityv == c).astype(
            jnp.float32)
    out_ref[64:68, :] = e_city
    out_ref[68:76, :] = jnp.maximum(
        dayw_ref[...] * d_ref[...] + dayb_ref[...], 0.0)
    out_ref[76:84, :] = jnp.maximum(
        timew_ref[:, 0:1] * ts_ref[...] + timew_ref[:, 1:2] * tc_ref[...]
        + timeb_ref[...], 0.0)
    out_ref[84:94, :] = jnp.maximum(
        jnp.dot(poiw_ref[...], poit_ref[...],
                preferred_element_type=jnp.float32) + poib_ref[...], 0.0)


def _tc_dense(e_uid, city1, d1, ts1, tc1, poit,
              citywt, dayw, dayb, timew, timeb, poiw, poib):
    bn = 2048
    grid = (_B // bn,)
    col = lambda i: (0, i)
    row = lambda i: (i, 0)
    rep = lambda i: (0, 0)
    return pl.pallas_call(
        _tc_body,
        grid=grid,
        in_specs=[
            pl.BlockSpec((bn, 128), row),
            pl.BlockSpec((1, bn), col),
            pl.BlockSpec((1, bn), col),
            pl.BlockSpec((1, bn), col),
            pl.BlockSpec((1, bn), col),
            pl.BlockSpec((85, bn), col),
            pl.BlockSpec((4, 4), rep),
            pl.BlockSpec((8, 1), rep),
            pl.BlockSpec((8, 1), rep),
            pl.BlockSpec((8, 2), rep),
            pl.BlockSpec((8, 1), rep),
            pl.BlockSpec((10, 85), rep),
            pl.BlockSpec((10, 1), rep),
        ],
        out_specs=pl.BlockSpec((94, bn), col),
        out_shape=jax.ShapeDtypeStruct((94, _B), jnp.float32),
    )(e_uid, city1, d1, ts1, tc1, poit,
      citywt, dayw, dayb, timew, timeb, poiw, poib)


def kernel(uid, d_norm, t_sin, t_cos, city, poi_norm,
           uid_emb_W, city_emb_W, day_W, day_b, time_W, time_b,
           poi_W, poi_b):
    table_rm = _tc_transpose(uid_emb_W.T)
    table3 = table_rm.reshape(_NUSERS // 8, 8, 128)
    e_uid = _make_sc_gather()(table3, uid.astype(jnp.int32))
    outt = _tc_dense(
        e_uid,
        city.astype(jnp.int32).reshape(1, _B),
        d_norm.reshape(1, _B),
        t_sin.reshape(1, _B),
        t_cos.reshape(1, _B),
        poi_norm.T,
        city_emb_W.T,
        day_W,
        day_b.reshape(8, 1),
        time_W,
        time_b.reshape(8, 1),
        poi_W,
        poi_b.reshape(10, 1),
    )
    return outt.T


# speedup vs baseline: 1.4031x; 1.0596x over previous
import functools

import jax
import jax.numpy as jnp
from jax import lax
from jax.experimental import pallas as pl
from jax.experimental.pallas import tpu as pltpu

try:
    from jax.experimental.pallas import tpu_sc as plsc
    _info = plsc.get_sparse_core_info()
    _NC, _NS = _info.num_cores, _info.num_subcores
except Exception:
    plsc = None
    _NC, _NS = 2, 16

_B = 16384
_UEMB = 64
_NUSERS = 1000000
_NW = _NC * _NS
_BPW = _B // _NW

_TN = 16384


def _transpose_body(int_ref, out_ref):
    t = int_ref[...].T
    out_ref[:, 0:_UEMB] = t[0:_TN // 2]
    out_ref[:, _UEMB:128] = t[_TN // 2:_TN]


def _tc_transpose(tabt):
    grid = (pl.cdiv(_NUSERS, _TN),)
    return pl.pallas_call(
        _transpose_body,
        grid=grid,
        in_specs=[pl.BlockSpec((_UEMB, _TN), lambda i: (0, i))],
        out_specs=pl.BlockSpec((_TN // 2, 128), lambda i: (i, 0)),
        out_shape=jax.ShapeDtypeStruct(
            (pl.cdiv(_NUSERS, _TN) * (_TN // 2), 128), jnp.float32),
    )(tabt)


_G = 16
_NGRP = _BPW // _G
_LAG = 2


def _make_sc_gather():
    mesh = plsc.VectorSubcoreMesh(core_axis_name="c", subcore_axis_name="s")

    @functools.partial(
        pl.kernel,
        mesh=mesh,
        out_type=jax.ShapeDtypeStruct((_B, 128), jnp.float32),
        scratch_types=[
            pltpu.VMEM((_BPW,), jnp.int32),
            pltpu.VMEM((_BPW, 128), jnp.float32),
            pltpu.SemaphoreType.DMA,
        ],
        compiler_params=pltpu.CompilerParams(use_tc_tiling_on_sc=True,
                                             needs_layout_passes=False),
    )
    def sc_gather(table_hbm, idx_hbm, out_hbm, idx_v, rows_v, sem):
        wid = lax.axis_index("s") * _NC + lax.axis_index("c")
        base = wid * _BPW
        lanes = lax.iota(jnp.int32, 16)
        pltpu.sync_copy(idx_hbm.at[pl.ds(base, _BPW)], idx_v)

        def fire(g):
            v = idx_v[pl.ds(g * _G, _G)]
            for j in range(_G):
                u = lax.reduce_max(jnp.where(lanes == j, v, -1), (0,))
                p = jnp.bitwise_or(
                    lax.shift_left(lax.shift_right_logical(u, 14), 13),
                    jnp.bitwise_and(u, 8191))
                t = lax.shift_right_logical(p, 3)
                s = jnp.bitwise_and(p, 7)
                pltpu.async_copy(table_hbm.at[t, s], rows_v.at[g * _G + j],
                                 sem)

        def drain():
            for j in range(_G):
                pltpu.make_async_copy(table_hbm.at[0, 0], rows_v.at[0],
                                      sem).wait()

        def body(g, carry):
            fire(g)

            @pl.when(g >= _LAG)
            def _():
                drain()

            return carry

        lax.fori_loop(0, _NGRP, body, 0)
        for _ in range(_LAG):
            drain()
        pltpu.sync_copy(rows_v, out_hbm.at[pl.ds(base, _BPW)])

    return sc_gather


def _tc_body(euid_ref, uidv_ref, city_ref, d_ref, ts_ref, tc_ref,
             poit_ref, citywt_ref, dayw_ref, dayb_ref, timew_ref,
             timeb_ref, poiw_ref, poib_ref, out_ref):
    h = jnp.bitwise_and(lax.shift_right_logical(uidv_ref[...], 13), 1)
    euid = jnp.where(h == 1, euid_ref[:, _UEMB:128], euid_ref[:, 0:_UEMB])
    out_ref[0:_UEMB, :] = euid.T
    cityv = city_ref[...]
    citywt = citywt_ref[...]
    e_city = citywt[:, 0:1] * (cityv == 0).astype(jnp.float32)
    for c in range(1, 4):
        e_city = e_city + citywt[:, c:c + 1] * (cityv == c).astype(
            jnp.float32)
    out_ref[64:68, :] = e_city
    out_ref[68:76, :] = jnp.maximum(
        dayw_ref[...] * d_ref[...] + dayb_ref[...], 0.0)
    out_ref[76:84, :] = jnp.maximum(
        timew_ref[:, 0:1] * ts_ref[...] + timew_ref[:, 1:2] * tc_ref[...]
        + timeb_ref[...], 0.0)
    out_ref[84:94, :] = jnp.maximum(
        jnp.dot(poiw_ref[...], poit_ref[...],
                preferred_element_type=jnp.float32) + poib_ref[...], 0.0)


def _tc_dense(e_uid, uid2, city1, d1, ts1, tc1, poit,
              citywt, dayw, dayb, timew, timeb, poiw, poib):
    bn = 2048
    grid = (_B // bn,)
    col = lambda i: (0, i)
    row = lambda i: (i, 0)
    rep = lambda i: (0, 0)
    return pl.pallas_call(
        _tc_body,
        grid=grid,
        in_specs=[
            pl.BlockSpec((bn, 128), row),
            pl.BlockSpec((bn, 1), row),
            pl.BlockSpec((1, bn), col),
            pl.BlockSpec((1, bn), col),
            pl.BlockSpec((1, bn), col),
            pl.BlockSpec((1, bn), col),
            pl.BlockSpec((85, bn), col),
            pl.BlockSpec((4, 4), rep),
            pl.BlockSpec((8, 1), rep),
            pl.BlockSpec((8, 1), rep),
            pl.BlockSpec((8, 2), rep),
            pl.BlockSpec((8, 1), rep),
            pl.BlockSpec((10, 85), rep),
            pl.BlockSpec((10, 1), rep),
        ],
        out_specs=pl.BlockSpec((94, bn), col),
        out_shape=jax.ShapeDtypeStruct((94, _B), jnp.float32),
    )(e_uid, uid2, city1, d1, ts1, tc1, poit,
      citywt, dayw, dayb, timew, timeb, poiw, poib)


def kernel(uid, d_norm, t_sin, t_cos, city, poi_norm,
           uid_emb_W, city_emb_W, day_W, day_b, time_W, time_b,
           poi_W, poi_b):
    table_pk = _tc_transpose(uid_emb_W.T)
    table3 = table_pk.reshape(table_pk.shape[0] // 8, 8, 128)
    e_uid = _make_sc_gather()(table3, uid.astype(jnp.int32))
    outt = _tc_dense(
        e_uid,
        uid.astype(jnp.int32).reshape(_B, 1),
        city.astype(jnp.int32).reshape(1, _B),
        d_norm.reshape(1, _B),
        t_sin.reshape(1, _B),
        t_cos.reshape(1, _B),
        poi_norm.T,
        city_emb_W.T,
        day_W,
        day_b.reshape(8, 1),
        time_W,
        time_b.reshape(8, 1),
        poi_W,
        poi_b.reshape(10, 1),
    )
    return outt.T
